# baseline (speedup 1.0000x reference)
import functools

import jax
import jax.numpy as jnp
from jax import lax
from jax.experimental import pallas as pl
from jax.experimental.pallas import tpu as pltpu
from jax.experimental.pallas import tpu_sc as plsc

_C = 40
_D = 256
_B = 16384
_BB = 4096
_NB = _B // _BB

_NC = 2
_NS = 16
_NW = _NC * _NS
_BPW = _B // _NW


def _gather_u(u_flat, idx):
    mesh = plsc.VectorSubcoreMesh(core_axis_name="c", subcore_axis_name="s")

    @functools.partial(
        pl.kernel,
        mesh=mesh,
        out_type=jax.ShapeDtypeStruct((_B,), jnp.float32),
        scratch_types=[
            pltpu.VMEM((_BPW,), jnp.int32),
            pltpu.VMEM((_BPW,), jnp.float32),
            pltpu.SemaphoreType.DMA,
        ],
    )
    def k(u_hbm, idx_hbm, out_hbm, idx_v, vals_v, sem):
        wid = lax.axis_index("s") * _NC + lax.axis_index("c")
        base = wid * _BPW
        pltpu.sync_copy(idx_hbm.at[pl.ds(base, _BPW)], idx_v)
        pltpu.async_copy(u_hbm.at[idx_v], vals_v, sem).wait()
        pltpu.sync_copy(vals_v, out_hbm.at[pl.ds(base, _BPW)])

    return k(u_flat, idx)


def _vexp(x):
    return jnp.max(jnp.exp(jnp.full((8, 128), x, dtype=jnp.float32)))


def _vlog(x):
    return jnp.max(jnp.log(jnp.full((8, 128), x, dtype=jnp.float32)))


def _sim_body(emb_ref, mv_ref, sim_ref):
    emb = emb_ref[...]
    mv = mv_ref[...]
    emb2 = emb * emb
    emb_ss = lax.dot_general(jnp.ones((1, _D), jnp.float32), emb2,
                             (((1,), (1,)), ((), ())),
                             preferred_element_type=jnp.float32)
    inv_en = lax.rsqrt(jnp.maximum(emb_ss, 1e-16))
    mv_ss = jnp.sum(mv * mv, axis=1, keepdims=True)
    inv_mn = lax.rsqrt(jnp.maximum(mv_ss, 1e-16))
    cos = lax.dot_general(mv, emb, (((1,), (1,)), ((), ())),
                          preferred_element_type=jnp.float32)
    sim_ref[...] = jnp.maximum(cos * inv_mn * inv_en, 0.0)


def _sim(emb, mv):
    return pl.pallas_call(
        _sim_body,
        grid=(_NB,),
        in_specs=[
            pl.BlockSpec((_BB, _D), lambda i: (i, 0)),
            pl.BlockSpec((_C, _D), lambda i: (0, 0)),
        ],
        out_specs=pl.BlockSpec((_C, _BB), lambda i: (0, i)),
        out_shape=jax.ShapeDtypeStruct((_C, _B), jnp.float32),
    )(emb, mv)


def _loss_body(ta_ref, logits_ref, label_ref, sim_ref, u_ref,
               o0_ref, o1_ref, o2_ref, o3_ref, acc_ref):
    i = pl.program_id(0)
    ta = ta_ref[0]
    logits = logits_ref[...]
    label = label_ref[...]
    sim = sim_ref[...] * label
    u_row = u_ref[...].reshape(1, _BB)

    lmax = jnp.max(logits, axis=0, keepdims=True)
    e = jnp.exp(logits - lmax)
    pred = e * (1.0 / jnp.sum(e, axis=0, keepdims=True))
    u_masked = u_row * label
    pred = jnp.clip(pred + ta * u_masked, 1e-4, 1.0)
    s1 = jnp.sum(sim * jnp.log(pred))

    rows = lax.broadcasted_iota(jnp.int32, (_C, _BB), 0)
    is_max = logits == lmax
    amax = jnp.min(jnp.where(is_max, rows, _C), axis=0, keepdims=True)
    onehot = (rows == amax).astype(jnp.float32)
    diff = onehot + u_masked - label
    s2 = jnp.sum(diff * diff)

    a = jnp.sum(logits * label, axis=0, keepdims=True)
    v = -jnp.log(jnp.maximum(u_row, 1e-8))
    mb_a = jnp.max(a)
    zb_a = jnp.sum(jnp.exp(a - mb_a))
    mb_v = jnp.max(v)
    ev = jnp.exp(v - mb_v)
    zb_v = jnp.sum(ev)
    wb = jnp.sum(ev * (v - a))

    @pl.when(i == 0)
    def _init():
        acc_ref[0] = s1
        acc_ref[1] = s2
        acc_ref[2] = mb_a
        acc_ref[3] = zb_a
        acc_ref[4] = mb_v
        acc_ref[5] = zb_v
        acc_ref[6] = wb

    @pl.when(i > 0)
    def _merge():
        acc_ref[0] += s1
        acc_ref[1] += s2
        m_a = acc_ref[2]
        nm_a = jnp.maximum(m_a, mb_a)
        acc_ref[3] = acc_ref[3] * _vexp(m_a - nm_a) + zb_a * _vexp(mb_a - nm_a)
        acc_ref[2] = nm_a
        m_v = acc_ref[4]
        nm_v = jnp.maximum(m_v, mb_v)
        sc_old = _vexp(m_v - nm_v)
        sc_new = _vexp(mb_v - nm_v)
        acc_ref[5] = acc_ref[5] * sc_old + zb_v * sc_new
        acc_ref[6] = acc_ref[6] * sc_old + wb * sc_new
        acc_ref[4] = nm_v

    @pl.when(i == _NB - 1)
    def _finalize():
        inv_b = 1.0 / _B
        l1 = -acc_ref[0] * inv_b
        l2 = acc_ref[1] * inv_b
        log_za = _vlog(acc_ref[3])
        log_zv = _vlog(acc_ref[5])
        kl = (acc_ref[6] / acc_ref[5]
              - (acc_ref[4] + log_zv) + (acc_ref[2] + log_za))
        l3 = (1.0 - ta) * kl * inv_b
        o0_ref[0] = l1 + l2 + l3
        o1_ref[0] = l1
        o2_ref[0] = l2
        o3_ref[0] = l3


def _loss(ta1, logits_t, label_t, sim_t, u_b):
    return pl.pallas_call(
        _loss_body,
        grid=(_NB,),
        in_specs=[
            pl.BlockSpec(memory_space=pltpu.SMEM),
            pl.BlockSpec((_C, _BB), lambda i: (0, i)),
            pl.BlockSpec((_C, _BB), lambda i: (0, i)),
            pl.BlockSpec((_C, _BB), lambda i: (0, i)),
            pl.BlockSpec((_BB,), lambda i: (i,)),
        ],
        out_specs=[pl.BlockSpec(memory_space=pltpu.SMEM)] * 4,
        out_shape=[jax.ShapeDtypeStruct((1,), jnp.float32)] * 4,
        scratch_shapes=[pltpu.SMEM((8,), jnp.float32)],
    )(ta1, logits_t, label_t, sim_t, u_b)


def kernel(batch_indices, model_logits, label_onehot, embeddings_detached,
           training_accuracy, u, prevSimilarity, masterVector):
    idx = batch_indices.astype(jnp.int32)
    u_flat = u.reshape(-1)
    u_b = _gather_u(u_flat, idx)
    sim_t = _sim(embeddings_detached, masterVector)
    o0, o1, o2, o3 = _loss(
        training_accuracy.reshape(1),
        model_logits.T,
        label_onehot.T,
        sim_t,
        u_b,
    )
    return (o0[0], o1[0], o2[0], o3[0])


# speedup vs baseline: 1.0124x; 1.0124x over previous
import functools

import jax
import jax.numpy as jnp
from jax import lax
from jax.experimental import pallas as pl
from jax.experimental.pallas import tpu as pltpu
from jax.experimental.pallas import tpu_sc as plsc

_C = 40
_D = 256
_B = 16384
_BB = 4096
_NB = _B // _BB

_NC = 2
_NS = 16
_NW = _NC * _NS
_BPW = _B // _NW


def _gather_u(u_flat, idx):
    mesh = plsc.VectorSubcoreMesh(core_axis_name="c", subcore_axis_name="s")

    @functools.partial(
        pl.kernel,
        mesh=mesh,
        out_type=jax.ShapeDtypeStruct((_B,), jnp.float32),
        scratch_types=[
            pltpu.VMEM((_BPW,), jnp.int32),
            pltpu.VMEM((_BPW,), jnp.float32),
            pltpu.SemaphoreType.DMA,
        ],
    )
    def k(u_hbm, idx_hbm, out_hbm, idx_v, vals_v, sem):
        wid = lax.axis_index("s") * _NC + lax.axis_index("c")
        base = wid * _BPW
        pltpu.sync_copy(idx_hbm.at[pl.ds(base, _BPW)], idx_v)
        pltpu.async_copy(u_hbm.at[idx_v], vals_v, sem).wait()
        pltpu.sync_copy(vals_v, out_hbm.at[pl.ds(base, _BPW)])

    return k(u_flat, idx)


def _vexp(x):
    return jnp.max(jnp.exp(jnp.full((8, 128), x, dtype=jnp.float32)))


def _vlog(x):
    return jnp.max(jnp.log(jnp.full((8, 128), x, dtype=jnp.float32)))


def _sim_body(emb_ref, mv_ref, sim_ref):
    emb = emb_ref[...]
    mv = mv_ref[...]
    emb2 = emb * emb
    emb_ss = lax.dot_general(jnp.ones((1, _D), jnp.float32), emb2,
                             (((1,), (1,)), ((), ())),
                             preferred_element_type=jnp.float32)
    inv_en = lax.rsqrt(jnp.maximum(emb_ss, 1e-16))
    mv_ss = jnp.sum(mv * mv, axis=1, keepdims=True)
    inv_mn = lax.rsqrt(jnp.maximum(mv_ss, 1e-16))
    cos = lax.dot_general(mv, emb, (((1,), (1,)), ((), ())),
                          preferred_element_type=jnp.float32)
    sim_ref[...] = jnp.maximum(cos * inv_mn * inv_en, 0.0).astype(jnp.bfloat16)


def _sim(emb, mv):
    return pl.pallas_call(
        _sim_body,
        grid=(_NB,),
        in_specs=[
            pl.BlockSpec((_BB, _D), lambda i: (i, 0)),
            pl.BlockSpec((_C, _D), lambda i: (0, 0)),
        ],
        out_specs=pl.BlockSpec((_C, _BB), lambda i: (0, i)),
        out_shape=jax.ShapeDtypeStruct((_C, _B), jnp.bfloat16),
    )(emb, mv)


def _loss_body(ta_ref, logits_ref, label_ref, sim_ref, u_ref,
               o0_ref, o1_ref, o2_ref, o3_ref, acc_ref):
    i = pl.program_id(0)
    ta = ta_ref[0]
    logits = logits_ref[...]
    label = label_ref[...]
    sim = sim_ref[...].astype(jnp.float32) * label
    u_row = u_ref[...].reshape(1, _BB)

    lmax = jnp.max(logits, axis=0, keepdims=True)
    e = jnp.exp(logits - lmax)
    pred = e * (1.0 / jnp.sum(e, axis=0, keepdims=True))
    u_masked = u_row * label
    pred = jnp.clip(pred + ta * u_masked, 1e-4, 1.0)
    s1 = jnp.sum(sim * jnp.log(pred))

    rows = lax.broadcasted_iota(jnp.int32, (_C, _BB), 0)
    is_max = logits == lmax
    amax = jnp.min(jnp.where(is_max, rows, _C), axis=0, keepdims=True)
    onehot = (rows == amax).astype(jnp.float32)
    diff = onehot + u_masked - label
    s2 = jnp.sum(diff * diff)

    a = jnp.sum(logits * label, axis=0, keepdims=True)
    v = -jnp.log(jnp.maximum(u_row, 1e-8))
    mb_a = jnp.max(a)
    zb_a = jnp.sum(jnp.exp(a - mb_a))
    mb_v = jnp.max(v)
    ev = jnp.exp(v - mb_v)
    zb_v = jnp.sum(ev)
    wb = jnp.sum(ev * (v - a))

    @pl.when(i == 0)
    def _init():
        acc_ref[0] = s1
        acc_ref[1] = s2
        acc_ref[2] = mb_a
        acc_ref[3] = zb_a
        acc_ref[4] = mb_v
        acc_ref[5] = zb_v
        acc_ref[6] = wb

    @pl.when(i > 0)
    def _merge():
        acc_ref[0] += s1
        acc_ref[1] += s2
        m_a = acc_ref[2]
        nm_a = jnp.maximum(m_a, mb_a)
        acc_ref[3] = acc_ref[3] * _vexp(m_a - nm_a) + zb_a * _vexp(mb_a - nm_a)
        acc_ref[2] = nm_a
        m_v = acc_ref[4]
        nm_v = jnp.maximum(m_v, mb_v)
        sc_old = _vexp(m_v - nm_v)
        sc_new = _vexp(mb_v - nm_v)
        acc_ref[5] = acc_ref[5] * sc_old + zb_v * sc_new
        acc_ref[6] = acc_ref[6] * sc_old + wb * sc_new
        acc_ref[4] = nm_v

    @pl.when(i == _NB - 1)
    def _finalize():
        inv_b = 1.0 / _B
        l1 = -acc_ref[0] * inv_b
        l2 = acc_ref[1] * inv_b
        log_za = _vlog(acc_ref[3])
        log_zv = _vlog(acc_ref[5])
        kl = (acc_ref[6] / acc_ref[5]
              - (acc_ref[4] + log_zv) + (acc_ref[2] + log_za))
        l3 = (1.0 - ta) * kl * inv_b
        o0_ref[0] = l1 + l2 + l3
        o1_ref[0] = l1
        o2_ref[0] = l2
        o3_ref[0] = l3


def _loss(ta1, logits_t, label_t, sim_t, u_b):
    return pl.pallas_call(
        _loss_body,
        grid=(_NB,),
        in_specs=[
            pl.BlockSpec(memory_space=pltpu.SMEM),
            pl.BlockSpec((_C, _BB), lambda i: (0, i)),
            pl.BlockSpec((_C, _BB), lambda i: (0, i)),
            pl.BlockSpec((_C, _BB), lambda i: (0, i)),
            pl.BlockSpec((_BB,), lambda i: (i,)),
        ],
        out_specs=[pl.BlockSpec(memory_space=pltpu.SMEM)] * 4,
        out_shape=[jax.ShapeDtypeStruct((1,), jnp.float32)] * 4,
        scratch_shapes=[pltpu.SMEM((8,), jnp.float32)],
    )(ta1, logits_t, label_t, sim_t, u_b)


def kernel(batch_indices, model_logits, label_onehot, embeddings_detached,
           training_accuracy, u, prevSimilarity, masterVector):
    idx = batch_indices.astype(jnp.int32)
    u_flat = u.reshape(-1)
    u_b = _gather_u(u_flat, idx)
    sim_t = _sim(embeddings_detached, masterVector)
    o0, o1, o2, o3 = _loss(
        training_accuracy.reshape(1),
        model_logits.T,
        label_onehot.T,
        sim_t,
        u_b,
    )
    return (o0[0], o1[0], o2[0], o3[0])


# speedup vs baseline: 1.0927x; 1.0793x over previous
import functools

import jax
import jax.numpy as jnp
from jax import lax
from jax.experimental import pallas as pl
from jax.experimental.pallas import tpu as pltpu
from jax.experimental.pallas import tpu_sc as plsc

_C = 40
_D = 256
_B = 16384
_BB = 4096
_NB = _B // _BB

_NC = 2
_NS = 16
_NW = _NC * _NS
_BPW = _B // _NW


def _gather_u(u_flat, idx):
    mesh = plsc.VectorSubcoreMesh(core_axis_name="c", subcore_axis_name="s")

    @functools.partial(
        pl.kernel,
        mesh=mesh,
        out_type=jax.ShapeDtypeStruct((_B,), jnp.float32),
        scratch_types=[
            pltpu.VMEM((_BPW,), jnp.int32),
            pltpu.VMEM((_BPW,), jnp.float32),
            pltpu.SemaphoreType.DMA,
        ],
    )
    def k(u_hbm, idx_hbm, out_hbm, idx_v, vals_v, sem):
        wid = lax.axis_index("s") * _NC + lax.axis_index("c")
        base = wid * _BPW
        pltpu.sync_copy(idx_hbm.at[pl.ds(base, _BPW)], idx_v)
        pltpu.async_copy(u_hbm.at[idx_v], vals_v, sem).wait()
        pltpu.sync_copy(vals_v, out_hbm.at[pl.ds(base, _BPW)])

    return k(u_flat, idx)


def _vexp(x):
    return jnp.max(jnp.exp(jnp.full((8, 128), x, dtype=jnp.float32)))


def _vlog(x):
    return jnp.max(jnp.log(jnp.full((8, 128), x, dtype=jnp.float32)))


def _dense_body(ta_ref, logits_ref, label_ref, emb_ref, mv_ref, u_ref,
                o0_ref, o1_ref, o2_ref, o3_ref, acc_ref):
    i = pl.program_id(0)
    ta = ta_ref[0]
    logits = logits_ref[...]
    label = label_ref[...]
    emb = emb_ref[...]
    mv = mv_ref[...]
    u_row = u_ref[...].reshape(1, _BB)

    emb2 = emb * emb
    emb_ss = lax.dot_general(jnp.ones((1, _D), jnp.float32), emb2,
                             (((1,), (1,)), ((), ())),
                             preferred_element_type=jnp.float32)
    inv_en = lax.rsqrt(jnp.maximum(emb_ss, 1e-16))
    mv_ss = jnp.sum(mv * mv, axis=1, keepdims=True)
    inv_mn = lax.rsqrt(jnp.maximum(mv_ss, 1e-16))
    sim = lax.dot_general(mv, emb, (((1,), (1,)), ((), ())),
                          preferred_element_type=jnp.float32)
    sim = jnp.maximum(sim * inv_mn * inv_en, 0.0) * label

    lmax = jnp.max(logits, axis=0, keepdims=True)
    e = jnp.exp(logits - lmax)
    pred = e * (1.0 / jnp.sum(e, axis=0, keepdims=True))
    u_masked = u_row * label
    pred = jnp.clip(pred + ta * u_masked, 1e-4, 1.0)
    s1 = jnp.sum(sim * jnp.log(pred))

    rows = lax.broadcasted_iota(jnp.int32, (_C, _BB), 0)
    is_max = logits == lmax
    amax = jnp.min(jnp.where(is_max, rows, _C), axis=0, keepdims=True)
    onehot = (rows == amax).astype(jnp.float32)
    diff = onehot + u_masked - label
    s2 = jnp.sum(diff * diff)

    a = jnp.sum(logits * label, axis=0, keepdims=True)
    v = -jnp.log(jnp.maximum(u_row, 1e-8))
    mb_a = jnp.max(a)
    zb_a = jnp.sum(jnp.exp(a - mb_a))
    mb_v = jnp.max(v)
    ev = jnp.exp(v - mb_v)
    zb_v = jnp.sum(ev)
    wb = jnp.sum(ev * (v - a))

    @pl.when(i == 0)
    def _init():
        acc_ref[0] = s1
        acc_ref[1] = s2
        acc_ref[2] = mb_a
        acc_ref[3] = zb_a
        acc_ref[4] = mb_v
        acc_ref[5] = zb_v
        acc_ref[6] = wb

    @pl.when(i > 0)
    def _merge():
        acc_ref[0] += s1
        acc_ref[1] += s2
        m_a = acc_ref[2]
        nm_a = jnp.maximum(m_a, mb_a)
        acc_ref[3] = acc_ref[3] * _vexp(m_a - nm_a) + zb_a * _vexp(mb_a - nm_a)
        acc_ref[2] = nm_a
        m_v = acc_ref[4]
        nm_v = jnp.maximum(m_v, mb_v)
        sc_old = _vexp(m_v - nm_v)
        sc_new = _vexp(mb_v - nm_v)
        acc_ref[5] = acc_ref[5] * sc_old + zb_v * sc_new
        acc_ref[6] = acc_ref[6] * sc_old + wb * sc_new
        acc_ref[4] = nm_v

    @pl.when(i == _NB - 1)
    def _finalize():
        inv_b = 1.0 / _B
        l1 = -acc_ref[0] * inv_b
        l2 = acc_ref[1] * inv_b
        log_za = _vlog(acc_ref[3])
        log_zv = _vlog(acc_ref[5])
        kl = (acc_ref[6] / acc_ref[5]
              - (acc_ref[4] + log_zv) + (acc_ref[2] + log_za))
        l3 = (1.0 - ta) * kl * inv_b
        o0_ref[0] = l1 + l2 + l3
        o1_ref[0] = l1
        o2_ref[0] = l2
        o3_ref[0] = l3


def _dense(ta1, logits_t, label_t, emb, mv, u_b):
    return pl.pallas_call(
        _dense_body,
        grid=(_NB,),
        in_specs=[
            pl.BlockSpec(memory_space=pltpu.SMEM),
            pl.BlockSpec((_C, _BB), lambda i: (0, i)),
            pl.BlockSpec((_C, _BB), lambda i: (0, i)),
            pl.BlockSpec((_BB, _D), lambda i: (i, 0)),
            pl.BlockSpec((_C, _D), lambda i: (0, 0)),
            pl.BlockSpec((_BB,), lambda i: (i,)),
        ],
        out_specs=[pl.BlockSpec(memory_space=pltpu.SMEM)] * 4,
        out_shape=[jax.ShapeDtypeStruct((1,), jnp.float32)] * 4,
        scratch_shapes=[pltpu.SMEM((8,), jnp.float32)],
    )(ta1, logits_t, label_t, emb, mv, u_b)


def kernel(batch_indices, model_logits, label_onehot, embeddings_detached,
           training_accuracy, u, prevSimilarity, masterVector):
    idx = batch_indices.astype(jnp.int32)
    u_b = _gather_u(u.reshape(-1), idx)
    o0, o1, o2, o3 = _dense(
        training_accuracy.reshape(1),
        model_logits.T,
        label_onehot.T,
        embeddings_detached,
        masterVector,
        u_b,
    )
    return (o0[0], o1[0], o2[0], o3[0])
